# Initial kernel scaffold; baseline (speedup 1.0000x reference)
import jax
import jax.numpy as jnp
from jax.experimental import pallas as pl


def kernel(x, edge_index, W_in, b_in, conv_w, W_out, b_out):
    raise NotImplementedError("write your pallas kernel here")


# speedup vs baseline: 5.3662x; 5.3662x over previous
import functools
import math

import jax
import jax.numpy as jnp
from jax import lax
from jax.experimental import pallas as pl
from jax.experimental.pallas import tpu as pltpu
from jax.experimental.pallas import tpu_sc as plsc

N = 10000
E = 160000
F = 256
HF = 128
NLAYERS = 16
A_RES = 0.1
TH_RES = 0.5

NC = 2
NS = 16
CH = 128
E_PAD = 163840
CPT = E_PAD // (NS * CH)
CPW = E_PAD // (NC * NS * CH)
RPT = 624
NACC = 10240
ZR = 128

BN = 1000

_sc_mesh = plsc.VectorSubcoreMesh(
    core_axis_name="c", subcore_axis_name="s", num_cores=NC, num_subcores=NS)


def _zero_fill(ref, nrows, val=0.0):
    v16 = jnp.full((16,), val, jnp.float32)

    @pl.loop(0, nrows)
    def _(r):
        for k in range(HF // 16):
            ref[r, pl.ds(k * 16, 16)] = v16


def _zero_acc(acc, zbuf, t):
    rows = NACC // NS
    for k in range(rows // ZR):
        pltpu.sync_copy(zbuf, acc.at[pl.ds(t * rows + k * ZR, ZR)])


@functools.partial(
    pl.kernel,
    out_type=jax.ShapeDtypeStruct((NC * NACC, HF), jnp.float32),
    mesh=_sc_mesh,
    scratch_types=[
        pltpu.VMEM_SHARED((NACC, HF), jnp.float32),
        pltpu.VMEM((CPW, CH), jnp.int32),
        pltpu.VMEM((CH, HF), jnp.float32),
    ],
)
def _sc_degree(dst_hbm, out_hbm, acc, dst_v, ones_v):
    c = lax.axis_index("c")
    t = lax.axis_index("s")
    w = c * NS + t
    _zero_fill(ones_v, CH)
    _zero_acc(acc, ones_v, t)
    _zero_fill(ones_v, CH, 1.0)
    pltpu.sync_copy(dst_hbm.at[w], dst_v)
    plsc.subcore_barrier()

    @pl.loop(0, CPW)
    def _(j):
        pltpu.sync_copy(ones_v, acc.at[dst_v.at[j]], add=True)

    plsc.subcore_barrier()
    rows = NACC // NS
    pltpu.sync_copy(acc.at[pl.ds(t * rows, rows)],
                    out_hbm.at[pl.ds(c * NACC + t * rows, rows)])


@functools.partial(
    pl.kernel,
    out_type=jax.ShapeDtypeStruct((NC * N, HF), jnp.float32),
    mesh=_sc_mesh,
    scratch_types=[
        pltpu.VMEM_SHARED((NACC, HF), jnp.float32),
        pltpu.VMEM((CPT // 2, CH), jnp.int32),
        pltpu.VMEM((CPT // 2, CH), jnp.int32),
        pltpu.VMEM((CH, HF), jnp.float32),
        pltpu.VMEM((CH, HF), jnp.float32),
        pltpu.SemaphoreType.DMA,
    ],
)
def _sc_propagate(hs_hbm, src_hbm, dst_hbm, out_hbm,
                  acc, src_v, dst_v, buf_a, buf_b, gsem):
    c = lax.axis_index("c")
    t = lax.axis_index("s")
    _zero_fill(buf_a, ZR)
    _zero_acc(acc, buf_a, t)
    plsc.subcore_barrier()

    hcp = CPT // 2
    for half in range(2):
        pltpu.sync_copy(src_hbm.at[(c * NS + t) * 2 + half], src_v)
        pltpu.sync_copy(dst_hbm.at[t * 2 + half], dst_v)

        pltpu.async_copy(hs_hbm.at[src_v.at[0]], buf_a, gsem)

        @pl.loop(0, hcp // 2)
        def _(i):
            j0 = 2 * i
            j1 = 2 * i + 1
            pltpu.make_async_copy(hs_hbm.at[src_v.at[j0]], buf_a, gsem).wait()
            pltpu.async_copy(hs_hbm.at[src_v.at[j1]], buf_b, gsem)
            pltpu.sync_copy(buf_a, acc.at[dst_v.at[j0]], add=True)
            pltpu.make_async_copy(hs_hbm.at[src_v.at[j1]], buf_b, gsem).wait()

            @pl.when(j1 + 1 < hcp)
            def _():
                pltpu.async_copy(hs_hbm.at[src_v.at[j1 + 1]], buf_a, gsem)

            pltpu.sync_copy(buf_b, acc.at[dst_v.at[j1]], add=True)

    plsc.subcore_barrier()
    row0 = t * RPT

    @pl.when(t < NS - 1)
    def _():
        pltpu.sync_copy(acc.at[pl.ds(row0, RPT)],
                        out_hbm.at[pl.ds(c * N + row0, RPT)])

    @pl.when(t == NS - 1)
    def _():
        last = N - (NS - 1) * RPT
        pltpu.sync_copy(acc.at[pl.ds(row0, last)],
                        out_hbm.at[pl.ds(c * N + row0, last)])


def _tc_input_body(x_ref, win_ref, bin_ref, deg_ref, x0_ref, hs_ref, dinv_ref):
    x0 = jnp.maximum(
        jnp.dot(x_ref[...], win_ref[...], preferred_element_type=jnp.float32)
        + bin_ref[...], 0.0)
    deg = deg_ref[0] + deg_ref[1] + 1.0
    dinv = lax.rsqrt(deg)
    x0a = x0[:, :HF]
    x0b = x0[:, HF:]
    x0_ref[0] = x0a
    x0_ref[1] = x0b
    hs_ref[0] = dinv * x0a
    hs_ref[1] = dinv * x0b
    dinv_ref[...] = dinv


def _tc_input(x, w_in, b_in, deg2):
    return pl.pallas_call(
        _tc_input_body,
        grid=(N // BN,),
        in_specs=[
            pl.BlockSpec((BN, F), lambda i: (i, 0)),
            pl.BlockSpec((F, F), lambda i: (0, 0)),
            pl.BlockSpec((1, F), lambda i: (0, 0)),
            pl.BlockSpec((NC, BN, 1), lambda i: (0, i, 0)),
        ],
        out_specs=[
            pl.BlockSpec((NC, BN, HF), lambda i: (0, i, 0)),
            pl.BlockSpec((NC, BN, HF), lambda i: (0, i, 0)),
            pl.BlockSpec((BN, 1), lambda i: (i, 0)),
        ],
        out_shape=[
            jax.ShapeDtypeStruct((NC, N, HF), jnp.float32),
            jax.ShapeDtypeStruct((NC, N, HF), jnp.float32),
            jax.ShapeDtypeStruct((N, 1), jnp.float32),
        ],
    )(x, w_in, b_in, deg2)


def _combine(hp_ref, hs_ref, x0_ref, dinv_ref, wl_ref, alpha, beta):
    d = dinv_ref[...]
    oa = (1.0 - alpha) * (d * (hp_ref[0] + hs_ref[0])) + alpha * x0_ref[0]
    ob = (1.0 - alpha) * (d * (hp_ref[1] + hs_ref[1])) + alpha * x0_ref[1]
    mm = (jnp.dot(oa, wl_ref[:HF, :], preferred_element_type=jnp.float32)
          + jnp.dot(ob, wl_ref[HF:, :], preferred_element_type=jnp.float32))
    ha = jnp.maximum((1.0 - beta) * oa + beta * mm[:, :HF], 0.0)
    hb = jnp.maximum((1.0 - beta) * ob + beta * mm[:, HF:], 0.0)
    return d, ha, hb


def _tc_layer_body(hp_ref, hs_ref, x0_ref, dinv_ref, wl_ref, out_ref, *,
                   alpha, beta):
    d, ha, hb = _combine(hp_ref, hs_ref, x0_ref, dinv_ref, wl_ref, alpha, beta)
    out_ref[0] = d * ha
    out_ref[1] = d * hb


def _tc_final_body(hp_ref, hs_ref, x0_ref, dinv_ref, wl_ref, wout_ref,
                   bout_ref, out_ref, *, alpha, beta):
    _, ha, hb = _combine(hp_ref, hs_ref, x0_ref, dinv_ref, wl_ref, alpha, beta)
    out_ref[...] = (
        jnp.dot(ha, wout_ref[:HF, :], preferred_element_type=jnp.float32)
        + jnp.dot(hb, wout_ref[HF:, :], preferred_element_type=jnp.float32)
        + bout_ref[...])


_layer_in_specs = [
    pl.BlockSpec((NC, BN, HF), lambda i: (0, i, 0)),
    pl.BlockSpec((NC, BN, HF), lambda i: (0, i, 0)),
    pl.BlockSpec((NC, BN, HF), lambda i: (0, i, 0)),
    pl.BlockSpec((BN, 1), lambda i: (i, 0)),
    pl.BlockSpec((F, F), lambda i: (0, 0)),
]


def _tc_layer(hp3, hsh, x0h, dinv, wl, alpha, beta):
    return pl.pallas_call(
        functools.partial(_tc_layer_body, alpha=alpha, beta=beta),
        grid=(N // BN,),
        in_specs=_layer_in_specs,
        out_specs=pl.BlockSpec((NC, BN, HF), lambda i: (0, i, 0)),
        out_shape=jax.ShapeDtypeStruct((NC, N, HF), jnp.float32),
    )(hp3, hsh, x0h, dinv, wl)


def _tc_final(hp3, hsh, x0h, dinv, wl, w_out, b_out, alpha, beta):
    return pl.pallas_call(
        functools.partial(_tc_final_body, alpha=alpha, beta=beta),
        grid=(N // BN,),
        in_specs=_layer_in_specs + [
            pl.BlockSpec((F, F), lambda i: (0, 0)),
            pl.BlockSpec((1, F), lambda i: (0, 0)),
        ],
        out_specs=pl.BlockSpec((BN, F), lambda i: (i, 0)),
        out_shape=jax.ShapeDtypeStruct((N, F), jnp.float32),
    )(hp3, hsh, x0h, dinv, wl, w_out, b_out)


def kernel(x, edge_index, W_in, b_in, conv_w, W_out, b_out):
    src = edge_index[0]
    dst = edge_index[1]
    pad = E_PAD - E
    src_p = jnp.concatenate([src, jnp.zeros((pad,), jnp.int32)])
    dst_p = jnp.concatenate([dst, jnp.full((pad,), N, jnp.int32)])
    src2 = jnp.stack([src_p, src_p + N]).reshape(NC * NS * 2, CPT // 2, CH)
    dst_prop = dst_p.reshape(NS * 2, CPT // 2, CH)
    dst_hist = dst_p.reshape(NC * NS, CPW, CH)

    hist = _sc_degree(dst_hist)
    deg2 = hist.reshape(NC, NACC, HF)[:, :N, :1]

    x0h, hsh, dinv = _tc_input(x, W_in, b_in.reshape(1, F), deg2)

    out = None
    for l in range(NLAYERS):
        beta = float(math.log(TH_RES / (l + 1) + 1.0))
        hp3 = _sc_propagate(hsh.reshape(NC * N, HF), src2, dst_prop)
        hp3 = hp3.reshape(NC, N, HF)
        if l < NLAYERS - 1:
            hsh = _tc_layer(hp3, hsh, x0h, dinv, conv_w[l], A_RES, beta)
        else:
            out = _tc_final(hp3, hsh, x0h, dinv, conv_w[l], W_out,
                            b_out.reshape(1, F), A_RES, beta)
    return out


# speedup vs baseline: 6.3428x; 1.1820x over previous
import functools
import math

import jax
import jax.numpy as jnp
from jax import lax
from jax.experimental import pallas as pl
from jax.experimental.pallas import tpu as pltpu
from jax.experimental.pallas import tpu_sc as plsc

N = 10000
E = 160000
F = 256
HF = 128
NLAYERS = 16
A_RES = 0.1
TH_RES = 0.5

NC = 2
NS = 16
CH = 128
GCH = 64
NBUF = 4
E_PAD = 163840
CPT = E_PAD // (NS * GCH)
NPASS = 4
CPP = CPT // NPASS
CPW = E_PAD // (NC * NS * CH)
RPT = 624
NACC = 10240
ZR = 64

BN = 1000

_sc_mesh = plsc.VectorSubcoreMesh(
    core_axis_name="c", subcore_axis_name="s", num_cores=NC, num_subcores=NS)


def _zero_fill(ref, nrows, val=0.0):
    v16 = jnp.full((16,), val, jnp.float32)

    @pl.loop(0, nrows)
    def _(r):
        for k in range(HF // 16):
            ref[r, pl.ds(k * 16, 16)] = v16


def _zero_acc(acc, zbuf, t):
    rows = NACC // NS
    for k in range(rows // ZR):
        pltpu.sync_copy(zbuf, acc.at[pl.ds(t * rows + k * ZR, ZR)])


@functools.partial(
    pl.kernel,
    out_type=jax.ShapeDtypeStruct((NC * NACC, HF), jnp.float32),
    mesh=_sc_mesh,
    scratch_types=[
        pltpu.VMEM_SHARED((NACC, HF), jnp.float32),
        pltpu.VMEM((CPW, CH), jnp.int32),
        pltpu.VMEM((CH, HF), jnp.float32),
    ],
)
def _sc_degree(dst_hbm, out_hbm, acc, dst_v, ones_v):
    c = lax.axis_index("c")
    t = lax.axis_index("s")
    w = c * NS + t
    _zero_fill(ones_v, ZR)
    _zero_acc(acc, ones_v.at[pl.ds(0, ZR)], t)
    _zero_fill(ones_v, CH, 1.0)
    pltpu.sync_copy(dst_hbm.at[w], dst_v)
    plsc.subcore_barrier()

    @pl.loop(0, CPW)
    def _(j):
        pltpu.sync_copy(ones_v, acc.at[dst_v.at[j]], add=True)

    plsc.subcore_barrier()
    rows = NACC // NS
    pltpu.sync_copy(acc.at[pl.ds(t * rows, rows)],
                    out_hbm.at[pl.ds(c * NACC + t * rows, rows)])


@functools.partial(
    pl.kernel,
    out_type=jax.ShapeDtypeStruct((NC * N, HF), jnp.float32),
    mesh=_sc_mesh,
    scratch_types=[
        pltpu.VMEM_SHARED((NACC, HF), jnp.float32),
        pltpu.VMEM((CPP, GCH), jnp.int32),
        pltpu.VMEM((CPP, GCH), jnp.int32),
        pltpu.VMEM((NBUF, GCH, HF), jnp.float32),
        [pltpu.SemaphoreType.DMA] * NBUF,
    ],
)
def _sc_propagate(hs_hbm, src_hbm, dst_hbm, out_hbm,
                  acc, src_v, dst_v, bufs, sems):
    c = lax.axis_index("c")
    t = lax.axis_index("s")
    _zero_fill(bufs.at[0], ZR)
    _zero_acc(acc, bufs.at[0].at[pl.ds(0, ZR)], t)
    plsc.subcore_barrier()

    hcp = CPP
    for half in range(NPASS):
        pltpu.sync_copy(src_hbm.at[(c * NS + t) * NPASS + half], src_v)
        pltpu.sync_copy(dst_hbm.at[t * NPASS + half], dst_v)

        for k in range(NBUF - 1):
            pltpu.async_copy(hs_hbm.at[src_v.at[k]], bufs.at[k], sems[k])

        @pl.loop(0, hcp // NBUF)
        def _(i):
            for k in range(NBUF):
                j = NBUF * i + k
                pltpu.make_async_copy(hs_hbm.at[src_v.at[j]], bufs.at[k],
                                      sems[k]).wait()
                kn = (k + NBUF - 1) % NBUF

                @pl.when(j + NBUF - 1 < hcp)
                def _():
                    pltpu.async_copy(hs_hbm.at[src_v.at[j + NBUF - 1]],
                                     bufs.at[kn], sems[kn])

                pltpu.sync_copy(bufs.at[k], acc.at[dst_v.at[j]], add=True)

    plsc.subcore_barrier()
    row0 = t * RPT

    @pl.when(t < NS - 1)
    def _():
        pltpu.sync_copy(acc.at[pl.ds(row0, RPT)],
                        out_hbm.at[pl.ds(c * N + row0, RPT)])

    @pl.when(t == NS - 1)
    def _():
        last = N - (NS - 1) * RPT
        pltpu.sync_copy(acc.at[pl.ds(row0, last)],
                        out_hbm.at[pl.ds(c * N + row0, last)])


def _tc_input_body(x_ref, win_ref, bin_ref, deg_ref, x0_ref, hs_ref, dinv_ref):
    x0 = jnp.maximum(
        jnp.dot(x_ref[...], win_ref[...], preferred_element_type=jnp.float32)
        + bin_ref[...], 0.0)
    deg = deg_ref[0] + deg_ref[1] + 1.0
    dinv = lax.rsqrt(deg)
    x0a = x0[:, :HF]
    x0b = x0[:, HF:]
    x0_ref[0] = x0a
    x0_ref[1] = x0b
    hs_ref[0] = dinv * x0a
    hs_ref[1] = dinv * x0b
    dinv_ref[...] = dinv


def _tc_input(x, w_in, b_in, deg2):
    return pl.pallas_call(
        _tc_input_body,
        grid=(N // BN,),
        in_specs=[
            pl.BlockSpec((BN, F), lambda i: (i, 0)),
            pl.BlockSpec((F, F), lambda i: (0, 0)),
            pl.BlockSpec((1, F), lambda i: (0, 0)),
            pl.BlockSpec((NC, BN, 1), lambda i: (0, i, 0)),
        ],
        out_specs=[
            pl.BlockSpec((NC, BN, HF), lambda i: (0, i, 0)),
            pl.BlockSpec((NC, BN, HF), lambda i: (0, i, 0)),
            pl.BlockSpec((BN, 1), lambda i: (i, 0)),
        ],
        out_shape=[
            jax.ShapeDtypeStruct((NC, N, HF), jnp.float32),
            jax.ShapeDtypeStruct((NC, N, HF), jnp.float32),
            jax.ShapeDtypeStruct((N, 1), jnp.float32),
        ],
    )(x, w_in, b_in, deg2)


def _combine(hp_ref, hs_ref, x0_ref, dinv_ref, wl_ref, alpha, beta):
    d = dinv_ref[...]
    oa = (1.0 - alpha) * (d * (hp_ref[0] + hs_ref[0])) + alpha * x0_ref[0]
    ob = (1.0 - alpha) * (d * (hp_ref[1] + hs_ref[1])) + alpha * x0_ref[1]
    mm = (jnp.dot(oa, wl_ref[:HF, :], preferred_element_type=jnp.float32)
          + jnp.dot(ob, wl_ref[HF:, :], preferred_element_type=jnp.float32))
    ha = jnp.maximum((1.0 - beta) * oa + beta * mm[:, :HF], 0.0)
    hb = jnp.maximum((1.0 - beta) * ob + beta * mm[:, HF:], 0.0)
    return d, ha, hb


def _tc_layer_body(hp_ref, hs_ref, x0_ref, dinv_ref, wl_ref, out_ref, *,
                   alpha, beta):
    d, ha, hb = _combine(hp_ref, hs_ref, x0_ref, dinv_ref, wl_ref, alpha, beta)
    out_ref[0] = d * ha
    out_ref[1] = d * hb


def _tc_final_body(hp_ref, hs_ref, x0_ref, dinv_ref, wl_ref, wout_ref,
                   bout_ref, out_ref, *, alpha, beta):
    _, ha, hb = _combine(hp_ref, hs_ref, x0_ref, dinv_ref, wl_ref, alpha, beta)
    out_ref[...] = (
        jnp.dot(ha, wout_ref[:HF, :], preferred_element_type=jnp.float32)
        + jnp.dot(hb, wout_ref[HF:, :], preferred_element_type=jnp.float32)
        + bout_ref[...])


_layer_in_specs = [
    pl.BlockSpec((NC, BN, HF), lambda i: (0, i, 0)),
    pl.BlockSpec((NC, BN, HF), lambda i: (0, i, 0)),
    pl.BlockSpec((NC, BN, HF), lambda i: (0, i, 0)),
    pl.BlockSpec((BN, 1), lambda i: (i, 0)),
    pl.BlockSpec((F, F), lambda i: (0, 0)),
]


def _tc_layer(hp3, hsh, x0h, dinv, wl, alpha, beta):
    return pl.pallas_call(
        functools.partial(_tc_layer_body, alpha=alpha, beta=beta),
        grid=(N // BN,),
        in_specs=_layer_in_specs,
        out_specs=pl.BlockSpec((NC, BN, HF), lambda i: (0, i, 0)),
        out_shape=jax.ShapeDtypeStruct((NC, N, HF), jnp.float32),
    )(hp3, hsh, x0h, dinv, wl)


def _tc_final(hp3, hsh, x0h, dinv, wl, w_out, b_out, alpha, beta):
    return pl.pallas_call(
        functools.partial(_tc_final_body, alpha=alpha, beta=beta),
        grid=(N // BN,),
        in_specs=_layer_in_specs + [
            pl.BlockSpec((F, F), lambda i: (0, 0)),
            pl.BlockSpec((1, F), lambda i: (0, 0)),
        ],
        out_specs=pl.BlockSpec((BN, F), lambda i: (i, 0)),
        out_shape=jax.ShapeDtypeStruct((N, F), jnp.float32),
    )(hp3, hsh, x0h, dinv, wl, w_out, b_out)


def kernel(x, edge_index, W_in, b_in, conv_w, W_out, b_out):
    src = edge_index[0]
    dst = edge_index[1]
    pad = E_PAD - E
    src_p = jnp.concatenate([src, jnp.zeros((pad,), jnp.int32)])
    dst_p = jnp.concatenate([dst, jnp.full((pad,), N, jnp.int32)])
    src2 = jnp.stack([src_p, src_p + N]).reshape(NC * NS * NPASS, CPP, GCH)
    dst_prop = dst_p.reshape(NS * NPASS, CPP, GCH)
    dst_hist = dst_p.reshape(NC * NS, CPW, CH)

    hist = _sc_degree(dst_hist)
    deg2 = hist.reshape(NC, NACC, HF)[:, :N, :1]

    x0h, hsh, dinv = _tc_input(x, W_in, b_in.reshape(1, F), deg2)

    out = None
    for l in range(NLAYERS):
        beta = float(math.log(TH_RES / (l + 1) + 1.0))
        hp3 = _sc_propagate(hsh.reshape(NC * N, HF), src2, dst_prop)
        hp3 = hp3.reshape(NC, N, HF)
        if l < NLAYERS - 1:
            hsh = _tc_layer(hp3, hsh, x0h, dinv, conv_w[l], A_RES, beta)
        else:
            out = _tc_final(hp3, hsh, x0h, dinv, conv_w[l], W_out,
                            b_out.reshape(1, F), A_RES, beta)
    return out


# speedup vs baseline: 9.4926x; 1.4966x over previous
import functools
import math

import jax
import jax.numpy as jnp
from jax import lax
from jax.experimental import pallas as pl
from jax.experimental.pallas import tpu as pltpu
from jax.experimental.pallas import tpu_sc as plsc

N = 10000
HN = N // 2
E = 160000
F = 256
HF = 128
NLAYERS = 16
A_RES = 0.1
TH_RES = 0.5

NC = 2
NS = 16
CH = 128
E_PAD = 163840
TOTC = E_PAD // CH
SEGC = 24
CPW = E_PAD // (NC * NS * CH)
NACC = 5120
TRASH = HN
NHIST = 10240
SRPT = 312

BN = 1000

_sc_mesh = plsc.VectorSubcoreMesh(
    core_axis_name="c", subcore_axis_name="s", num_cores=NC, num_subcores=NS)


def _fill(ref, nrows, val):
    v16 = jnp.full((16,), val, jnp.float32)

    @pl.loop(0, nrows)
    def _(r):
        for k in range(HF // 16):
            ref[r, pl.ds(k * 16, 16)] = v16


def _split5k(src_ref, src0, dst_ref, dst0, t):
    @pl.when(t < NS - 1)
    def _():
        pltpu.sync_copy(src_ref.at[pl.ds(src0, SRPT)],
                        dst_ref.at[pl.ds(dst0, SRPT)])

    @pl.when(t == NS - 1)
    def _():
        last = HN - (NS - 1) * SRPT
        pltpu.sync_copy(src_ref.at[pl.ds(src0, last)],
                        dst_ref.at[pl.ds(dst0, last)])


@functools.partial(
    pl.kernel,
    out_type=jax.ShapeDtypeStruct((NC * NHIST, HF), jnp.float32),
    mesh=_sc_mesh,
    scratch_types=[
        pltpu.VMEM_SHARED((NHIST, HF), jnp.float32),
        pltpu.VMEM((CPW, CH), jnp.int32),
        pltpu.VMEM((CH, HF), jnp.float32),
    ],
)
def _sc_degree(dst_hbm, out_hbm, acc, dst_v, ones_v):
    c = lax.axis_index("c")
    t = lax.axis_index("s")
    w = c * NS + t
    rows = NHIST // NS
    _fill(ones_v, CH, 0.0)
    for k in range(rows // CH):
        pltpu.sync_copy(ones_v, acc.at[pl.ds(t * rows + k * CH, CH)])
    _fill(ones_v, CH, 1.0)
    pltpu.sync_copy(dst_hbm.at[w], dst_v)
    plsc.subcore_barrier()

    @pl.loop(0, CPW)
    def _(j):
        pltpu.sync_copy(ones_v, acc.at[dst_v.at[j]], add=True)

    plsc.subcore_barrier()
    pltpu.sync_copy(acc.at[pl.ds(t * rows, rows)],
                    out_hbm.at[pl.ds(c * NHIST + t * rows, rows)])


@functools.partial(
    pl.kernel,
    out_type=jax.ShapeDtypeStruct((NC * N, HF), jnp.float32),
    mesh=_sc_mesh,
    scratch_types=[
        pltpu.VMEM_SHARED((NACC, HF), jnp.float32),
        pltpu.VMEM_SHARED((HN, HF), jnp.float32),
        pltpu.VMEM((SEGC * 2, CH), jnp.int32),
        pltpu.VMEM((CH, HF), jnp.float32),
        pltpu.VMEM((CH, HF), jnp.float32),
        pltpu.VMEM((2, 16), jnp.int32),
        [pltpu.SemaphoreType.DMA] * 2,
    ],
)
def _sc_propagate(hs_hbm, comb_hbm, param_hbm, out_hbm,
                  acc, table, iseg, buf_a, buf_b, pbuf, sems):
    c = lax.axis_index("c")
    t = lax.axis_index("s")

    for d in range(2):
        _fill(buf_a, CH, 0.0)
        for bi in range(3):
            blk = t + NS * bi

            @pl.when(blk < NACC // CH)
            def _():
                pltpu.sync_copy(buf_a, acc.at[pl.ds(blk * CH, CH)])

        for s in range(2):
            b = 2 * d + s
            _split5k(hs_hbm, c * N + s * HN + SRPT * t, table, SRPT * t, t)
            pltpu.sync_copy(param_hbm.at[b * NS + t], pbuf)
            plsc.subcore_barrier()
            start = pbuf[0, pl.ds(0, 16)][0]
            cnt = pbuf[1, pl.ds(0, 16)][0]
            nseg = (cnt + SEGC - 1) // SEGC

            @pl.loop(0, nseg)
            def _(sg):
                off = pl.multiple_of((start + SEGC * sg) * 2, 8)
                pltpu.sync_copy(comb_hbm.at[pl.ds(off, SEGC * 2)], iseg)
                scnt = jnp.minimum(cnt - SEGC * sg, SEGC)

                @pl.when(scnt > 0)
                def _():
                    pltpu.async_copy(table.at[iseg.at[0]], buf_a, sems[0])

                @pl.loop(0, (scnt + 1) // 2)
                def _(i):
                    j0 = 2 * i
                    j1 = 2 * i + 1

                    @pl.when(j0 < scnt)
                    def _():
                        pltpu.make_async_copy(table.at[iseg.at[2 * j0]],
                                              buf_a, sems[0]).wait()

                    @pl.when(j1 < scnt)
                    def _():
                        pltpu.async_copy(table.at[iseg.at[2 * j1]],
                                         buf_b, sems[1])

                    @pl.when(j0 < scnt)
                    def _():
                        pltpu.sync_copy(buf_a, acc.at[iseg.at[2 * j0 + 1]],
                                        add=True)

                    @pl.when(j1 < scnt)
                    def _():
                        pltpu.make_async_copy(table.at[iseg.at[2 * j1]],
                                              buf_b, sems[1]).wait()

                    @pl.when(j1 + 1 < scnt)
                    def _():
                        pltpu.async_copy(table.at[iseg.at[2 * (j1 + 1)]],
                                         buf_a, sems[0])

                    @pl.when(j1 < scnt)
                    def _():
                        pltpu.sync_copy(buf_b, acc.at[iseg.at[2 * j1 + 1]],
                                        add=True)

            plsc.subcore_barrier()

        _split5k(acc, SRPT * t, out_hbm, c * N + d * HN + SRPT * t, t)
        plsc.subcore_barrier()


def _tc_input_body(x_ref, win_ref, bin_ref, deg_ref, x0_ref, hs_ref, dinv_ref):
    x0 = jnp.maximum(
        jnp.dot(x_ref[...], win_ref[...], preferred_element_type=jnp.float32)
        + bin_ref[...], 0.0)
    deg = deg_ref[0] + deg_ref[1] + 1.0
    dinv = lax.rsqrt(deg)
    x0a = x0[:, :HF]
    x0b = x0[:, HF:]
    x0_ref[0] = x0a
    x0_ref[1] = x0b
    hs_ref[0] = dinv * x0a
    hs_ref[1] = dinv * x0b
    dinv_ref[...] = dinv


def _tc_input(x, w_in, b_in, deg2):
    return pl.pallas_call(
        _tc_input_body,
        grid=(N // BN,),
        in_specs=[
            pl.BlockSpec((BN, F), lambda i: (i, 0)),
            pl.BlockSpec((F, F), lambda i: (0, 0)),
            pl.BlockSpec((1, F), lambda i: (0, 0)),
            pl.BlockSpec((NC, BN, 1), lambda i: (0, i, 0)),
        ],
        out_specs=[
            pl.BlockSpec((NC, BN, HF), lambda i: (0, i, 0)),
            pl.BlockSpec((NC, BN, HF), lambda i: (0, i, 0)),
            pl.BlockSpec((BN, 1), lambda i: (i, 0)),
        ],
        out_shape=[
            jax.ShapeDtypeStruct((NC, N, HF), jnp.float32),
            jax.ShapeDtypeStruct((NC, N, HF), jnp.float32),
            jax.ShapeDtypeStruct((N, 1), jnp.float32),
        ],
    )(x, w_in, b_in, deg2)


def _combine(hp_ref, hs_ref, x0_ref, dinv_ref, wl_ref, alpha, beta):
    d = dinv_ref[...]
    oa = (1.0 - alpha) * (d * (hp_ref[0] + hs_ref[0])) + alpha * x0_ref[0]
    ob = (1.0 - alpha) * (d * (hp_ref[1] + hs_ref[1])) + alpha * x0_ref[1]
    mm = (jnp.dot(oa, wl_ref[:HF, :], preferred_element_type=jnp.float32)
          + jnp.dot(ob, wl_ref[HF:, :], preferred_element_type=jnp.float32))
    ha = jnp.maximum((1.0 - beta) * oa + beta * mm[:, :HF], 0.0)
    hb = jnp.maximum((1.0 - beta) * ob + beta * mm[:, HF:], 0.0)
    return d, ha, hb


def _tc_layer_body(hp_ref, hs_ref, x0_ref, dinv_ref, wl_ref, out_ref, *,
                   alpha, beta):
    d, ha, hb = _combine(hp_ref, hs_ref, x0_ref, dinv_ref, wl_ref, alpha, beta)
    out_ref[0] = d * ha
    out_ref[1] = d * hb


def _tc_final_body(hp_ref, hs_ref, x0_ref, dinv_ref, wl_ref, wout_ref,
                   bout_ref, out_ref, *, alpha, beta):
    _, ha, hb = _combine(hp_ref, hs_ref, x0_ref, dinv_ref, wl_ref, alpha, beta)
    out_ref[...] = (
        jnp.dot(ha, wout_ref[:HF, :], preferred_element_type=jnp.float32)
        + jnp.dot(hb, wout_ref[HF:, :], preferred_element_type=jnp.float32)
        + bout_ref[...])


_layer_in_specs = [
    pl.BlockSpec((NC, BN, HF), lambda i: (0, i, 0)),
    pl.BlockSpec((NC, BN, HF), lambda i: (0, i, 0)),
    pl.BlockSpec((NC, BN, HF), lambda i: (0, i, 0)),
    pl.BlockSpec((BN, 1), lambda i: (i, 0)),
    pl.BlockSpec((F, F), lambda i: (0, 0)),
]


def _tc_layer(hp3, hsh, x0h, dinv, wl, alpha, beta):
    return pl.pallas_call(
        functools.partial(_tc_layer_body, alpha=alpha, beta=beta),
        grid=(N // BN,),
        in_specs=_layer_in_specs,
        out_specs=pl.BlockSpec((NC, BN, HF), lambda i: (0, i, 0)),
        out_shape=jax.ShapeDtypeStruct((NC, N, HF), jnp.float32),
    )(hp3, hsh, x0h, dinv, wl)


def _tc_final(hp3, hsh, x0h, dinv, wl, w_out, b_out, alpha, beta):
    return pl.pallas_call(
        functools.partial(_tc_final_body, alpha=alpha, beta=beta),
        grid=(N // BN,),
        in_specs=_layer_in_specs + [
            pl.BlockSpec((F, F), lambda i: (0, 0)),
            pl.BlockSpec((1, F), lambda i: (0, 0)),
        ],
        out_specs=pl.BlockSpec((BN, F), lambda i: (i, 0)),
        out_shape=jax.ShapeDtypeStruct((N, F), jnp.float32),
    )(hp3, hsh, x0h, dinv, wl, w_out, b_out)


def _edge_setup(edge_index):
    src = edge_index[0]
    dst = edge_index[1]
    pad = E_PAD - E
    src_p = jnp.concatenate([src, jnp.zeros((pad,), jnp.int32)])
    dst_p = jnp.concatenate([dst, jnp.full((pad,), N, jnp.int32)])
    bucket = jnp.minimum((dst_p // HN) * 2 + src_p // HN, 3)
    order = jnp.argsort(bucket, stable=True)
    src_s = src_p[order]
    dst_s = dst_p[order]
    bkt_s = bucket[order]
    src_l = src_s % HN
    combs = []
    for b in range(4):
        dst_l = jnp.where((bkt_s == b) & (dst_s < N), dst_s % HN, TRASH)
        combs.append(jnp.stack(
            [src_l.reshape(TOTC, CH), dst_l.reshape(TOTC, CH)], axis=1))
    comb = jnp.concatenate(combs, axis=0)
    trash_rows = jnp.broadcast_to(
        jnp.stack([jnp.zeros((CH,), jnp.int32),
                   jnp.full((CH,), TRASH, jnp.int32)]), (4 * SEGC, 2, CH))
    comb = jnp.concatenate([comb, trash_rows], axis=0).reshape(-1, CH)

    edges = jnp.searchsorted(bkt_s, jnp.arange(5, dtype=jnp.int32))
    lane = jnp.arange(NS, dtype=jnp.int32)
    params = []
    for b in range(4):
        u0 = edges[b] // (4 * CH)
        u1 = (edges[b + 1] + 4 * CH - 1) // (4 * CH)
        nu = u1 - u0
        base = nu // NS
        rem = nu % NS
        cnts = 4 * (base + (lane < rem).astype(jnp.int32))
        starts = b * TOTC + 4 * (u0 + lane * base + jnp.minimum(lane, rem))
        params.append(jnp.stack([starts, cnts]))
    param = jnp.stack(params).astype(jnp.int32)
    param = param.transpose(0, 2, 1).reshape(4 * NS, 2, 1)
    param = jnp.pad(param, ((0, 0), (0, 0), (0, 15)))
    dst_hist = dst_p.reshape(NC * NS, CPW, CH)
    return comb, param, dst_hist


def kernel(x, edge_index, W_in, b_in, conv_w, W_out, b_out):
    comb, param, dst_hist = _edge_setup(edge_index)

    hist = _sc_degree(dst_hist)
    deg2 = hist.reshape(NC, NHIST, HF)[:, :N, :1]

    x0h, hsh, dinv = _tc_input(x, W_in, b_in.reshape(1, F), deg2)

    out = None
    for l in range(NLAYERS):
        beta = float(math.log(TH_RES / (l + 1) + 1.0))
        hp3 = _sc_propagate(hsh.reshape(NC * N, HF), comb, param)
        hp3 = hp3.reshape(NC, N, HF)
        if l < NLAYERS - 1:
            hsh = _tc_layer(hp3, hsh, x0h, dinv, conv_w[l], A_RES, beta)
        else:
            out = _tc_final(hp3, hsh, x0h, dinv, conv_w[l], W_out,
                            b_out.reshape(1, F), A_RES, beta)
    return out
